# Initial kernel scaffold; baseline (speedup 1.0000x reference)
import jax
import jax.numpy as jnp
from jax.experimental import pallas as pl


def kernel(x, edge_index, W1, b1, W2, b2, W3, b3, Wl1, bl1, Wl2, bl2):
    raise NotImplementedError("write your pallas kernel here")


# speedup vs baseline: 10.3244x; 10.3244x over previous
import functools

import jax
import jax.numpy as jnp
from jax import lax
from jax.experimental import pallas as pl
from jax.experimental.pallas import tpu as pltpu
from jax.experimental.pallas import tpu_sc as plsc

NNODE = 100000
NEDGE = 3200000

NC = 2
NS = 16
NW = NC * NS

RSH = 14
RS = 1 << RSH
P = (NNODE + RS - 1) >> RSH
SPAD = RS + 128
NPAD = NNODE + 16
NOUT = P * RS

ET = NEDGE // NW
CH = 2000
FLUSH = 512
STG = FLUSH + 16
NBLK_SEG = ET // FLUSH + 1
SEG = NBLK_SEG * FLUSH
TOTE = NW * P * SEG
NBLKTOT = NW * P * NBLK_SEG
SZCH = 344
WL = 544
DUMMY_BLK = NBLKTOT

_mesh = plsc.VectorSubcoreMesh(core_axis_name="c", subcore_axis_name="s")


def _prefix16(x, iota):
    y = x
    for d in (1, 2, 4, 8):
        idx = jnp.maximum(iota - d, 0)
        sh = y.at[idx].get(mode="promise_in_bounds")
        y = y + jnp.where(iota >= d, sh, 0)
    return y


def _build_worklist(pv, s, cntf, worklist, iota):
    cr0 = cntf[pl.ds(pv * NW, 16)]
    cr1 = cntf[pl.ds(pv * NW + 16, 16)]
    wcnt = jnp.int32(0)
    gbase = jnp.int32(0)
    for t2 in range(NW):
        v = cr0 if t2 < 16 else cr1
        nb = v[t2 % 16]
        b0 = lax.rem(s - gbase, jnp.int32(16))
        b0 = jnp.where(b0 < 0, b0 + 16, b0)
        nmy = jnp.maximum((nb - b0 + 15) // 16, 0)
        cand = (t2 * P) * NBLK_SEG + pv * NBLK_SEG + b0 + iota * 16
        plsc.store_scatter(worklist, [wcnt + iota], cand, mask=iota < nmy)
        wcnt = wcnt + nmy
        gbase = gbase + nb
    npad = lax.rem(jnp.int32(16) - lax.rem(wcnt, jnp.int32(16)), jnp.int32(16))
    plsc.store_scatter(worklist, [wcnt + iota],
                       jnp.full((16,), DUMMY_BLK, jnp.int32),
                       mask=iota < npad)
    return (wcnt + npad) // 16


def _pp_count():
    return (P + NC - 1) // NC


def _bin_body(rows_hbm, cols_hbm, brow_hbm, bcol_hbm, counts_hbm,
              rowch, colch, stg_r, stg_c, cntbuf, scnt):
    c = lax.axis_index("c")
    s = lax.axis_index("s")
    t = c * NS + s
    e0 = t * ET
    iota = lax.iota(jnp.int32, 16)
    drow = jnp.int32(NNODE) + iota
    dcol = jnp.int32(RS) + iota

    for p in range(P):
        scnt[p] = jnp.int32(0)
        scnt[8 + p] = jnp.int32(0)

    def chunk_body(k, carry):
        pltpu.sync_copy(rows_hbm.at[pl.ds(e0 + k * CH, CH)], rowch)
        pltpu.sync_copy(cols_hbm.at[pl.ds(e0 + k * CH, CH)], colch)

        def vec_body(v, carry2):
            r16 = rowch[pl.ds(v * 16, 16)]
            c16 = colch[pl.ds(v * 16, 16)]
            p16 = lax.shift_right_logical(c16, RSH)
            l16 = lax.bitwise_and(c16, RS - 1)
            for p in range(P):
                msk = p16 == p
                cnt = scnt[p]
                pc = _prefix16(jnp.where(msk, jnp.int32(1), jnp.int32(0)), iota)
                dest = cnt + pc - 1
                plsc.store_scatter(stg_r[p], [dest], r16, mask=msk)
                plsc.store_scatter(stg_c[p], [dest], l16, mask=msk)
                npop = pc[15]
                cnt2 = cnt + npop
                do_flush = cnt2 >= FLUSH

                @pl.when(do_flush)
                def _():
                    nb = scnt[8 + p]
                    base = (t * P + p) * SEG + nb * FLUSH
                    pltpu.sync_copy(stg_r[p].at[pl.ds(0, FLUSH)],
                                    brow_hbm.at[pl.ds(base, FLUSH)])
                    pltpu.sync_copy(stg_c[p].at[pl.ds(0, FLUSH)],
                                    bcol_hbm.at[pl.ds(base, FLUSH)])
                    rem = cnt2 - FLUSH
                    pm = iota < rem
                    tr = stg_r[p][pl.ds(FLUSH, 16)]
                    tcv = stg_c[p][pl.ds(FLUSH, 16)]
                    plsc.store_scatter(stg_r[p], [iota], tr, mask=pm)
                    plsc.store_scatter(stg_c[p], [iota], tcv, mask=pm)
                    scnt[8 + p] = nb + 1

                scnt[p] = jnp.where(do_flush, cnt2 - FLUSH, cnt2)
            return carry2

        lax.fori_loop(0, CH // 16, vec_body, 0)
        return carry

    lax.fori_loop(0, ET // CH, chunk_body, 0)

    cvec = jnp.zeros((16,), jnp.int32)
    for p in range(P):
        cnt = scnt[p]

        def fill_body(j, carry):
            idx16 = j * 16 + iota
            m = idx16 >= cnt
            cur_r = stg_r[p][pl.ds(j * 16, 16)]
            cur_c = stg_c[p][pl.ds(j * 16, 16)]
            stg_r[p][pl.ds(j * 16, 16)] = jnp.where(m, drow, cur_r)
            stg_c[p][pl.ds(j * 16, 16)] = jnp.where(m, dcol, cur_c)
            return carry

        lax.fori_loop(0, FLUSH // 16, fill_body, 0)
        nb = scnt[8 + p]

        @pl.when(cnt > 0)
        def _():
            base = (t * P + p) * SEG + nb * FLUSH
            pltpu.sync_copy(stg_r[p].at[pl.ds(0, FLUSH)],
                            brow_hbm.at[pl.ds(base, FLUSH)])
            pltpu.sync_copy(stg_c[p].at[pl.ds(0, FLUSH)],
                            bcol_hbm.at[pl.ds(base, FLUSH)])

        nbf = jnp.where(cnt > 0, nb + 1, nb)
        cvec = jnp.where(iota == p, nbf, cvec)

    cntbuf[...] = cvec
    pltpu.sync_copy(cntbuf, counts_hbm.at[t])

    @pl.when(t == 0)
    def _():
        def fillall(j, carry):
            stg_r[0][pl.ds(j * 16, 16)] = drow
            stg_c[0][pl.ds(j * 16, 16)] = dcol
            return carry

        lax.fori_loop(0, FLUSH // 16, fillall, 0)
        pltpu.sync_copy(stg_r[0].at[pl.ds(0, FLUSH)],
                        brow_hbm.at[pl.ds(NW * P * SEG, FLUSH)])
        pltpu.sync_copy(stg_c[0].at[pl.ds(0, FLUSH)],
                        bcol_hbm.at[pl.ds(NW * P * SEG, FLUSH)])


def _bin_edges(rows, cols):
    k = pl.kernel(
        _bin_body,
        out_type=[
            jax.ShapeDtypeStruct((TOTE + FLUSH,), jnp.int32),
            jax.ShapeDtypeStruct((TOTE + FLUSH,), jnp.int32),
            jax.ShapeDtypeStruct((NW, 16), jnp.int32),
        ],
        mesh=_mesh,
        compiler_params=pltpu.CompilerParams(needs_layout_passes=False, use_tc_tiling_on_sc=False),
        scratch_types=[
            pltpu.VMEM((CH,), jnp.int32),
            pltpu.VMEM((CH,), jnp.int32),
            [pltpu.VMEM((STG,), jnp.int32) for _ in range(P)],
            [pltpu.VMEM((STG,), jnp.int32) for _ in range(P)],
            pltpu.VMEM((16,), jnp.int32),
            pltpu.SMEM((16,), jnp.int32),
        ],
    )
    return k(rows, cols)


def _deg_body(bcol_hbm, counts_hbm, deg_hbm,
              colb, ones, zbuf, cntf, worklist, deg_sp, sema):
    c = lax.axis_index("c")
    s = lax.axis_index("s")
    iota = lax.iota(jnp.int32, 16)
    pltpu.sync_copy(counts_hbm, cntf)

    def zb(i, carry):
        zbuf[pl.ds(i * 16, 16)] = jnp.zeros((16,), jnp.float32)
        return carry

    lax.fori_loop(0, SZCH // 16, zb, 0)

    def ob(i, carry):
        ones[pl.ds(i * 16, 16)] = jnp.ones((16,), jnp.float32)
        return carry

    lax.fori_loop(0, 128 // 16, ob, 0)

    for pp in range(_pp_count()):
        pv = pp * NC + c

        @pl.when(pv < P)
        def _():
            for q in range(3):
                pltpu.sync_copy(zbuf, deg_sp.at[pl.ds((s * 3 + q) * SZCH, SZCH)])
            ngrp = _build_worklist(pv, s, cntf, worklist, iota)
            plsc.subcore_barrier()

            def grp_body(g, carry):
                wv = worklist[pl.ds(g * 16, 16)]
                for j in range(16):
                    blk = wv[j]
                    pltpu.sync_copy(bcol_hbm.at[blk], colb)
                    ds_ = [pltpu.async_copy(ones, deg_sp.at[colb.at[jj]],
                                            sema, add=True) for jj in range(4)]
                    for d in ds_:
                        d.wait()
                return carry

            lax.fori_loop(0, ngrp, grp_body, 0)
            plsc.subcore_barrier()
            pltpu.sync_copy(deg_sp.at[pl.ds(s * 1024, 1024)],
                            deg_hbm.at[pl.ds(pv * RS + s * 1024, 1024)])
            plsc.subcore_barrier()


def _degrees(bcol_blk, counts_flat):
    k = pl.kernel(
        _deg_body,
        out_type=[jax.ShapeDtypeStruct((NOUT,), jnp.float32)],
        mesh=_mesh,
        compiler_params=pltpu.CompilerParams(needs_layout_passes=False, use_tc_tiling_on_sc=False),
        scratch_types=[
            pltpu.VMEM((4, 128), jnp.int32),
            pltpu.VMEM((128,), jnp.float32),
            pltpu.VMEM((SZCH,), jnp.float32),
            pltpu.VMEM((P * NW,), jnp.int32),
            pltpu.VMEM((WL,), jnp.int32),
            pltpu.VMEM_SHARED((SPAD,), jnp.float32),
            pltpu.SemaphoreType.DMA,
        ],
    )
    (deg,) = k(bcol_blk, counts_flat)
    return deg


def _acc_body(F, y_hbm, brow_hbm, bcol_hbm, counts_hbm, s_hbm,
              rowb, colb, msg, zbuf, cntf, worklist, s_sp, semg, sema):
    c = lax.axis_index("c")
    s = lax.axis_index("s")
    iota = lax.iota(jnp.int32, 16)
    pltpu.sync_copy(counts_hbm, cntf)

    def zb(r, carry):
        for cc in range(F // 16):
            zbuf[r, pl.ds(cc * 16, 16)] = jnp.zeros((16,), jnp.float32)
        return carry

    lax.fori_loop(0, SZCH, zb, 0)

    for pp in range(_pp_count()):
        pv = pp * NC + c

        @pl.when(pv < P)
        def _():
            for q in range(3):
                pltpu.sync_copy(zbuf, s_sp.at[pl.ds((s * 3 + q) * SZCH, SZCH)])
            ngrp = _build_worklist(pv, s, cntf, worklist, iota)
            plsc.subcore_barrier()

            def grp_body(g, carry):
                wv = worklist[pl.ds(g * 16, 16)]
                for j in range(16):
                    blk = wv[j]
                    pltpu.sync_copy(brow_hbm.at[blk], rowb)
                    pltpu.sync_copy(bcol_hbm.at[blk], colb)
                    gd = [pltpu.async_copy(y_hbm.at[rowb.at[jj]],
                                           msg.at[pl.ds(jj * 128, 128)], semg)
                          for jj in range(4)]
                    for d in gd:
                        d.wait()
                    sd = [pltpu.async_copy(msg.at[pl.ds(jj * 128, 128)],
                                           s_sp.at[colb.at[jj]], sema, add=True)
                          for jj in range(4)]
                    for d in sd:
                        d.wait()
                return carry

            lax.fori_loop(0, ngrp, grp_body, 0)
            plsc.subcore_barrier()
            pltpu.sync_copy(s_sp.at[pl.ds(s * 1024, 1024)],
                            s_hbm.at[pl.ds(pv * RS + s * 1024, 1024)])
            plsc.subcore_barrier()


def _accumulate(y, brow_blk, bcol_blk, counts_flat, F):
    k = pl.kernel(
        functools.partial(_acc_body, F),
        out_type=[jax.ShapeDtypeStruct((NOUT, F), jnp.float32)],
        mesh=_mesh,
        compiler_params=pltpu.CompilerParams(needs_layout_passes=False, use_tc_tiling_on_sc=False),
        scratch_types=[
            pltpu.VMEM((4, 128), jnp.int32),
            pltpu.VMEM((4, 128), jnp.int32),
            pltpu.VMEM((512, F), jnp.float32),
            pltpu.VMEM((SZCH, F), jnp.float32),
            pltpu.VMEM((P * NW,), jnp.int32),
            pltpu.VMEM((WL,), jnp.int32),
            pltpu.VMEM_SHARED((SPAD, F), jnp.float32),
            pltpu.SemaphoreType.DMA,
            pltpu.SemaphoreType.DMA,
        ],
    )
    (out,) = k(y, brow_blk, bcol_blk, counts_flat)
    return out


BR = 1024
GRID = (NPAD + BR - 1) // BR


def _tc_first(deg2, x, W1):
    def f(deg_ref, x_ref, w_ref, dis_ref, y_ref):
        dis = lax.rsqrt(deg_ref[...] + 1.0)
        dis_ref[...] = dis
        y_ref[...] = jnp.dot(x_ref[...], w_ref[...],
                             preferred_element_type=jnp.float32) * dis

    return pl.pallas_call(
        f,
        grid=(GRID,),
        in_specs=[
            pl.BlockSpec((BR, 1), lambda i: (i, 0)),
            pl.BlockSpec((BR, 21), lambda i: (i, 0)),
            pl.BlockSpec((21, 32), lambda i: (0, 0)),
        ],
        out_specs=[
            pl.BlockSpec((BR, 1), lambda i: (i, 0)),
            pl.BlockSpec((BR, 32), lambda i: (i, 0)),
        ],
        out_shape=[
            jax.ShapeDtypeStruct((NPAD, 1), jnp.float32),
            jax.ShapeDtypeStruct((NPAD, 32), jnp.float32),
        ],
    )(deg2, x, W1)


def _tc_mid(sarr, y, dis, b, W, Fin, Fout):
    def f(s_ref, y_ref, d_ref, b_ref, w_ref, o_ref):
        d = d_ref[...]
        h = jnp.maximum(d * (s_ref[...] + y_ref[...]) + b_ref[...], 0.0)
        o_ref[...] = jnp.dot(h, w_ref[...],
                             preferred_element_type=jnp.float32) * d

    return pl.pallas_call(
        f,
        grid=(GRID,),
        in_specs=[
            pl.BlockSpec((BR, Fin), lambda i: (i, 0)),
            pl.BlockSpec((BR, Fin), lambda i: (i, 0)),
            pl.BlockSpec((BR, 1), lambda i: (i, 0)),
            pl.BlockSpec((1, Fin), lambda i: (0, 0)),
            pl.BlockSpec((Fin, Fout), lambda i: (0, 0)),
        ],
        out_specs=pl.BlockSpec((BR, Fout), lambda i: (i, 0)),
        out_shape=jax.ShapeDtypeStruct((NPAD, Fout), jnp.float32),
    )(sarr, y, dis, b, W)


def _tc_head(s3, y3, dis, b3, Wl1, bl1, Wl2, bl2):
    def f(s_ref, y_ref, d_ref, b_ref, w1_ref, c1_ref, w2_ref, c2_ref, o_ref):
        d = d_ref[...]
        h = jnp.maximum(d * (s_ref[...] + y_ref[...]) + b_ref[...], 0.0)
        z = jnp.maximum(jnp.dot(h, w1_ref[...],
                                preferred_element_type=jnp.float32)
                        + c1_ref[...], 0.0)
        o_ref[...] = jnp.dot(z, w2_ref[...],
                             preferred_element_type=jnp.float32) + c2_ref[...]

    return pl.pallas_call(
        f,
        grid=(GRID,),
        in_specs=[
            pl.BlockSpec((BR, 32), lambda i: (i, 0)),
            pl.BlockSpec((BR, 32), lambda i: (i, 0)),
            pl.BlockSpec((BR, 1), lambda i: (i, 0)),
            pl.BlockSpec((1, 32), lambda i: (0, 0)),
            pl.BlockSpec((32, 20), lambda i: (0, 0)),
            pl.BlockSpec((1, 20), lambda i: (0, 0)),
            pl.BlockSpec((20, 1), lambda i: (0, 0)),
            pl.BlockSpec((1, 1), lambda i: (0, 0)),
        ],
        out_specs=pl.BlockSpec((BR, 1), lambda i: (i, 0)),
        out_shape=jax.ShapeDtypeStruct((NNODE, 1), jnp.float32),
    )(s3, y3, dis, b3, Wl1, bl1, Wl2, bl2)


def kernel(x, edge_index, W1, b1, W2, b2, W3, b3, Wl1, bl1, Wl2, bl2):
    rows = edge_index[0]
    cols = edge_index[1]

    brow, bcol, counts = _bin_edges(rows, cols)
    brow_blk = brow.reshape(NBLKTOT + 1, 4, 128)
    bcol_blk = bcol.reshape(NBLKTOT + 1, 4, 128)
    counts_flat = counts[:, :P].T.reshape(P * NW)

    deg = _degrees(bcol_blk, counts_flat)

    dis, y1 = _tc_first(deg.reshape(NOUT, 1), x, W1)
    s1 = _accumulate(y1, brow_blk, bcol_blk, counts_flat, 32)
    y2 = _tc_mid(s1, y1, dis, b1.reshape(1, -1), W2, 32, 64)
    s2 = _accumulate(y2, brow_blk, bcol_blk, counts_flat, 64)
    y3 = _tc_mid(s2, y2, dis, b2.reshape(1, -1), W3, 64, 32)
    s3 = _accumulate(y3, brow_blk, bcol_blk, counts_flat, 32)
    return _tc_head(s3, y3, dis, b3.reshape(1, -1), Wl1, bl1.reshape(1, -1),
                    Wl2, bl2.reshape(1, -1))
